# baseline (speedup 1.0000x reference)
import functools

import jax
import jax.numpy as jnp
from jax.experimental import pallas as pl
from jax.experimental.pallas import tpu as pltpu

_CHUNK = 1024


def _full_spec(shape):
    nd = len(shape)
    return pl.BlockSpec(shape, lambda *_, _nd=nd: (0,) * _nd)


def _pool_stacked(h2, w3_ref, nb, final_relu):
    L = h2.shape[1]
    per = max(L // _CHUNK, 1)
    cs = L // per
    ms = []
    for i in range(nb):
        slab = h2[128 * i:128 * (i + 1)]
        m = None
        for j in range(per):
            h3 = jax.lax.dot_general(slab[:, j * cs:(j + 1) * cs], w3_ref[...],
                                     (((0,), (0,)), ((), ())),
                                     preferred_element_type=jnp.float32)
            mj = jnp.max(h3, axis=0, keepdims=True)
            m = mj if m is None else jnp.maximum(m, mj)
        if final_relu:
            m = jnp.maximum(m, 0.0)
        ms.append(m)
    return jnp.concatenate(ms, axis=0)[:, None]


def _stn_stack_kernel(x_ref, w1bd_ref, w2bd_ref, w3_ref, o_ref):
    nb = o_ref.shape[0]
    xs = x_ref[0]
    h1 = jnp.dot(w1bd_ref[...], xs, preferred_element_type=jnp.float32)
    h1 = jnp.maximum(h1, 0.0).astype(jnp.bfloat16)
    h2 = jnp.dot(w2bd_ref[...], h1, preferred_element_type=jnp.float32)
    h2 = jnp.maximum(h2, 0.0).astype(jnp.bfloat16)
    o_ref[...] = _pool_stacked(h2, w3_ref, nb, final_relu=True)


def _feat_stack_kernel(x_ref, c_ref, w1xbd_ref, w1cbd_ref, w2bd_ref, w3_ref,
                       o_ref):
    nb = o_ref.shape[0]
    xs = x_ref[0]
    cs = c_ref[...] if c_ref.ndim == 2 else c_ref[0]
    h1 = jnp.dot(w1xbd_ref[0], xs, preferred_element_type=jnp.float32)
    h1 = h1 + jnp.dot(w1cbd_ref[...], cs, preferred_element_type=jnp.float32)
    h1 = jnp.maximum(h1, 0.0).astype(jnp.bfloat16)
    h2 = jnp.dot(w2bd_ref[...], h1, preferred_element_type=jnp.float32)
    h2 = jnp.maximum(h2, 0.0).astype(jnp.bfloat16)
    o_ref[...] = _pool_stacked(h2, w3_ref, nb, final_relu=False)


def _prep_kernel(nw, nb, x_ref, *refs):
    for i in range(nw):
        refs[nw + 1 + i][...] = refs[i][...].astype(jnp.bfloat16)
    xg_ref = refs[nw]
    n = x_ref.shape[1]
    groups = xg_ref.shape[0]
    for g in range(groups):
        for i in range(nb):
            xg_ref[g, n * i:n * (i + 1), :] = \
                x_ref[g * nb + i].astype(jnp.bfloat16)


def _prep(x, nb, weights):
    B, n, L = x.shape
    nw = len(weights)
    out = pl.pallas_call(
        functools.partial(_prep_kernel, nw, nb),
        out_shape=([jax.ShapeDtypeStruct((B // nb, nb * n, L), jnp.bfloat16)]
                   + [jax.ShapeDtypeStruct(w.shape, jnp.bfloat16)
                      for w in weights]),
        grid=(1,),
        in_specs=[_full_spec(x.shape)] + [_full_spec(w.shape)
                                          for w in weights],
        out_specs=([_full_spec((B // nb, nb * n, L))]
                   + [_full_spec(w.shape) for w in weights]),
        compiler_params=pltpu.CompilerParams(
            dimension_semantics=("arbitrary",)),
    )(x, *weights)
    return out[0], out[1:]


def _stn_head_kernel(g_ref, fw1_ref, fw2_ref, fw3_ref, fb3_ref, fx_ref,
                     o_ref, w1x_ref):
    g = g_ref[...].astype(jnp.bfloat16)
    g = jnp.dot(g, fw1_ref[...].astype(jnp.bfloat16),
                preferred_element_type=jnp.float32)
    g = jnp.maximum(g, 0.0).astype(jnp.bfloat16)
    g = jnp.dot(g, fw2_ref[...].astype(jnp.bfloat16),
                preferred_element_type=jnp.float32)
    g = jnp.maximum(g, 0.0).astype(jnp.bfloat16)
    g = jnp.dot(g, fw3_ref[...].astype(jnp.bfloat16),
                preferred_element_type=jnp.float32) + fb3_ref[...]
    o_ref[...] = g
    n = fx_ref.shape[0]
    for i in range(n):
        wi = jnp.dot(g[:, n * i:n * (i + 1)], fx_ref[...],
                     preferred_element_type=jnp.float32)
        w1x_ref[:, i, :] = wi.astype(jnp.bfloat16)


def _cls_head_kernel(g_ref, w1_ref, b1_ref, w2_ref, b2_ref, w3_ref, b3_ref,
                     o_ref):
    g = g_ref[...].astype(jnp.bfloat16)
    g = jnp.dot(g, w1_ref[...].astype(jnp.bfloat16),
                preferred_element_type=jnp.float32)
    g = jnp.maximum(g + b1_ref[...], 0.0).astype(jnp.bfloat16)
    g = jnp.dot(g, w2_ref[...].astype(jnp.bfloat16),
                preferred_element_type=jnp.float32)
    g = jnp.maximum(g + b2_ref[...], 0.0).astype(jnp.bfloat16)
    logits = jnp.dot(g, w3_ref[...].astype(jnp.bfloat16),
                     preferred_element_type=jnp.float32) + b3_ref[...]
    mx = jnp.max(logits, axis=0, keepdims=True)
    lse = mx + jnp.log(jnp.sum(jnp.exp(logits - mx), axis=0, keepdims=True))
    o_ref[...] = logits - lse


def kernel(x, c, stn_w1, stn_w2, stn_w3, stn_fw1, stn_fw2, stn_fw3, stn_fb3,
           f_w1x, f_w1c, f_w2, f_w3, c_w1, c_b1, c_w2, c_b2, c_w3, c_b3):
    B, n, L = x.shape
    ne = c.shape[1]
    k = c_w3.shape[1]

    nb = 4
    if B % nb != 0 or L % 1024 != 0 or L > 4096:
        nb = 1
    if L % 256 != 0:
        Lp = -(-L // 256) * 256
        x = jnp.pad(x, ((0, 0), (0, 0), (0, Lp - L)), mode="edge")
        c = jnp.pad(c, ((0, 0), (0, 0), (0, Lp - L)), mode="edge")
        L = Lp

    bf = lambda a: a.astype(jnp.bfloat16)
    eye = jnp.eye(nb, dtype=jnp.float32)
    kron = lambda w: jnp.kron(eye, w)

    xg, (w1bd_b, w2bd_b, w3_b, w1cbd_b, fw2bd_b, fw3_b) = _prep(
        x, nb, [kron(stn_w1.T), kron(stn_w2.T), stn_w3,
                kron(f_w1c.T), kron(f_w2.T), f_w3])

    flat_c = (nb * ne) % 8 == 0
    c2 = bf(c).reshape(B * ne, L) if flat_c else bf(c)

    cparams_pool = pltpu.CompilerParams(
        dimension_semantics=("parallel",),
        vmem_limit_bytes=100 * 2**20)
    cparams_head = pltpu.CompilerParams(dimension_semantics=("arbitrary",))

    pool_out_shape = jax.ShapeDtypeStruct((B, 1, 1024), jnp.float32)
    pool_out_spec = pl.BlockSpec((nb, 1, 1024), lambda b: (b, 0, 0))
    x_spec = pl.BlockSpec((1, nb * n, L), lambda b: (b, 0, 0))
    if flat_c:
        c_spec = pl.BlockSpec((nb * ne, L), lambda b: (b, 0))
    else:
        c_spec = pl.BlockSpec((nb, ne, L), lambda b: (b, 0, 0))
    w3_spec = _full_spec((128, 1024))
    grid = (B // nb,)

    g1 = pl.pallas_call(
        _stn_stack_kernel,
        out_shape=pool_out_shape,
        grid=grid,
        in_specs=[x_spec, _full_spec((nb * 64, nb * n)),
                  _full_spec((nb * 128, nb * 64)), w3_spec],
        out_specs=pool_out_spec,
        compiler_params=cparams_pool,
    )(xg, w1bd_b, w2bd_b, w3_b)
    g1 = g1.reshape(B, 1024)

    trans_flat, w1x_eff = pl.pallas_call(
        _stn_head_kernel,
        out_shape=[jax.ShapeDtypeStruct((B, n * n), jnp.float32),
                   jax.ShapeDtypeStruct((B, n, 64), jnp.bfloat16)],
        grid=(1,),
        in_specs=[_full_spec((B, 1024)), _full_spec((1024, 512)),
                  _full_spec((512, 256)), _full_spec((256, n * n)),
                  _full_spec((1, n * n)), _full_spec((n, 64))],
        out_specs=[_full_spec((B, n * n)), _full_spec((B, n, 64))],
        compiler_params=cparams_head,
    )(g1, stn_fw1, stn_fw2, stn_fw3, stn_fb3, f_w1x)
    trans = trans_flat.reshape(B, n, n)

    w1x_bd = jnp.einsum(
        "ij,gjla->giajl", eye.astype(jnp.bfloat16),
        w1x_eff.reshape(B // nb, nb, n, 64)).reshape(B // nb, nb * 64, nb * n)

    g2 = pl.pallas_call(
        _feat_stack_kernel,
        out_shape=pool_out_shape,
        grid=grid,
        in_specs=[x_spec, c_spec,
                  pl.BlockSpec((1, nb * 64, nb * n), lambda b: (b, 0, 0)),
                  _full_spec((nb * 64, nb * ne)),
                  _full_spec((nb * 128, nb * 64)), w3_spec],
        out_specs=pool_out_spec,
        compiler_params=cparams_pool,
    )(xg, c2, w1x_bd, w1cbd_b, fw2bd_b, fw3_b)
    g2 = g2.reshape(B, 1024)

    logp = pl.pallas_call(
        _cls_head_kernel,
        out_shape=jax.ShapeDtypeStruct((B, k), jnp.float32),
        grid=(1,),
        in_specs=[_full_spec((B, 1024)),
                  _full_spec((1024, 512)), _full_spec((1, 512)),
                  _full_spec((512, 256)), _full_spec((1, 256)),
                  _full_spec((256, k)), _full_spec((1, k))],
        out_specs=_full_spec((B, k)),
        compiler_params=cparams_head,
    )(g2, c_w1, c_b1, c_w2, c_b2, c_w3, c_b3)

    return logp, trans


# speedup vs baseline: 1.0658x; 1.0658x over previous
import functools

import jax
import jax.numpy as jnp
from jax.experimental import pallas as pl
from jax.experimental.pallas import tpu as pltpu

_CHUNK = 1024


def _pool_chunks(h2, w3_ref, nb, final_relu):
    tl = h2.shape[1] // nb
    per = max(tl // _CHUNK, 1)
    cs = tl // per
    ms = []
    for b in range(nb):
        m = None
        for j in range(per):
            lo = b * tl + j * cs
            h3 = jax.lax.dot_general(h2[:, lo:lo + cs], w3_ref[...],
                                     (((0,), (0,)), ((), ())),
                                     preferred_element_type=jnp.float32)
            mj = jnp.max(h3, axis=0, keepdims=True)
            m = mj if m is None else jnp.maximum(m, mj)
        if final_relu:
            m = jnp.maximum(m, 0.0)
        ms.append(m)
    return jnp.concatenate(ms, axis=0)[:, None]


def _lane_cat(ref, cast=True):
    parts = [ref[i] for i in range(ref.shape[0])]
    out = parts[0] if len(parts) == 1 else jnp.concatenate(parts, axis=1)
    return out.astype(jnp.bfloat16) if cast else out


def _stn_stack_kernel(x_ref, w1t_ref, w2t_ref, w3_ref, o_ref):
    nb = x_ref.shape[0]
    xb = _lane_cat(x_ref)
    h1 = jnp.dot(w1t_ref[...], xb, preferred_element_type=jnp.float32)
    h1 = jnp.maximum(h1, 0.0).astype(jnp.bfloat16)
    h2 = jnp.dot(w2t_ref[...], h1, preferred_element_type=jnp.float32)
    h2 = jnp.maximum(h2, 0.0).astype(jnp.bfloat16)
    o_ref[...] = _pool_chunks(h2, w3_ref, nb, final_relu=True)


def _feat_stack_kernel(x_ref, c_ref, w1x_ref, w1ct_ref, w2t_ref, w3_ref,
                       o_ref):
    nb = x_ref.shape[0]
    cb = _lane_cat(c_ref)
    h1x = [jax.lax.dot_general(w1x_ref[i], x_ref[i].astype(jnp.bfloat16),
                               (((0,), (0,)), ((), ())),
                               preferred_element_type=jnp.float32)
           for i in range(nb)]
    h1x = h1x[0] if nb == 1 else jnp.concatenate(h1x, axis=1)
    h1 = h1x + jnp.dot(w1ct_ref[...], cb, preferred_element_type=jnp.float32)
    h1 = jnp.maximum(h1, 0.0).astype(jnp.bfloat16)
    h2 = jnp.dot(w2t_ref[...], h1, preferred_element_type=jnp.float32)
    h2 = jnp.maximum(h2, 0.0).astype(jnp.bfloat16)
    o_ref[...] = _pool_chunks(h2, w3_ref, nb, final_relu=False)


def _prep_kernel(*refs):
    nio = len(refs) // 2
    for i in range(nio):
        refs[nio + i][...] = refs[i][...].astype(jnp.bfloat16)


def _prep_bf16(arrays):
    return pl.pallas_call(
        _prep_kernel,
        out_shape=[jax.ShapeDtypeStruct(a.shape, jnp.bfloat16)
                   for a in arrays],
        grid=(1,),
        in_specs=[_full_spec(a.shape) for a in arrays],
        out_specs=[_full_spec(a.shape) for a in arrays],
        compiler_params=pltpu.CompilerParams(
            dimension_semantics=("arbitrary",)),
    )(*arrays)


def _stn_head_kernel(g_ref, fw1_ref, fw2_ref, fw3_ref, fb3_ref, fx_ref,
                     o_ref, w1x_ref):
    g = g_ref[...].astype(jnp.bfloat16)
    g = jnp.dot(g, fw1_ref[...].astype(jnp.bfloat16),
                preferred_element_type=jnp.float32)
    g = jnp.maximum(g, 0.0).astype(jnp.bfloat16)
    g = jnp.dot(g, fw2_ref[...].astype(jnp.bfloat16),
                preferred_element_type=jnp.float32)
    g = jnp.maximum(g, 0.0).astype(jnp.bfloat16)
    g = jnp.dot(g, fw3_ref[...].astype(jnp.bfloat16),
                preferred_element_type=jnp.float32) + fb3_ref[...]
    o_ref[...] = g
    n = fx_ref.shape[0]
    for i in range(n):
        wi = jnp.dot(g[:, n * i:n * (i + 1)], fx_ref[...],
                     preferred_element_type=jnp.float32)
        w1x_ref[:, i, :] = wi.astype(jnp.bfloat16)


def _cls_head_kernel(g_ref, w1_ref, b1_ref, w2_ref, b2_ref, w3_ref, b3_ref,
                     o_ref):
    g = g_ref[...].astype(jnp.bfloat16)
    g = jnp.dot(g, w1_ref[...].astype(jnp.bfloat16),
                preferred_element_type=jnp.float32)
    g = jnp.maximum(g + b1_ref[...], 0.0).astype(jnp.bfloat16)
    g = jnp.dot(g, w2_ref[...].astype(jnp.bfloat16),
                preferred_element_type=jnp.float32)
    g = jnp.maximum(g + b2_ref[...], 0.0).astype(jnp.bfloat16)
    logits = jnp.dot(g, w3_ref[...].astype(jnp.bfloat16),
                     preferred_element_type=jnp.float32) + b3_ref[...]
    mx = jnp.max(logits, axis=0, keepdims=True)
    lse = mx + jnp.log(jnp.sum(jnp.exp(logits - mx), axis=0, keepdims=True))
    o_ref[...] = logits - lse


def _full_spec(shape):
    nd = len(shape)
    return pl.BlockSpec(shape, lambda *_, _nd=nd: (0,) * _nd)


def kernel(x, c, stn_w1, stn_w2, stn_w3, stn_fw1, stn_fw2, stn_fw3, stn_fb3,
           f_w1x, f_w1c, f_w2, f_w3, c_w1, c_b1, c_w2, c_b2, c_w3, c_b3):
    B, n, L = x.shape
    ne = c.shape[1]
    k = c_w3.shape[1]

    tl = min(L, 4096)
    if L % tl != 0:
        num = -(-L // tl)
        Lp = num * tl
        x = jnp.pad(x, ((0, 0), (0, 0), (0, Lp - L)), mode="edge")
        c = jnp.pad(c, ((0, 0), (0, 0), (0, Lp - L)), mode="edge")
        L = Lp
    num_lt = L // tl

    nb = 4 if (num_lt == 1 and B % 4 == 0) else 1

    cparams_pool = pltpu.CompilerParams(
        dimension_semantics=("parallel",) if num_lt == 1
        else ("parallel", "arbitrary"),
        vmem_limit_bytes=100 * 2**20)
    cparams_head = pltpu.CompilerParams(dimension_semantics=("arbitrary",))

    w1t_b, w2t_b, w3_b, fw1ct_b, fw2t_b, fw3_b = _prep_bf16(
        [stn_w1.T, stn_w2.T, stn_w3, f_w1c.T, f_w2.T, f_w3])

    pool_out_shape = jax.ShapeDtypeStruct((B, 1, 1024), jnp.float32)
    if num_lt == 1:
        grid = (B // nb,)
        pool_out_spec = pl.BlockSpec((nb, 1, 1024), lambda b: (b, 0, 0))
        x_spec = pl.BlockSpec((nb, n, tl), lambda b: (b, 0, 0))
        c_spec = pl.BlockSpec((nb, ne, tl), lambda b: (b, 0, 0))
        w1x_spec = pl.BlockSpec((nb, n, 64), lambda b: (b, 0, 0))
    else:
        grid = (B, num_lt)
        pool_out_spec = pl.BlockSpec((1, 1, 1024), lambda b, lt: (b, 0, 0))
        x_spec = pl.BlockSpec((1, n, tl), lambda b, lt: (b, 0, lt))
        c_spec = pl.BlockSpec((1, ne, tl), lambda b, lt: (b, 0, lt))
        w1x_spec = pl.BlockSpec((1, n, 64), lambda b, lt: (b, 0, 0))
    w2t_spec = _full_spec((128, 64))
    w3_spec = _full_spec((128, 1024))

    stn_stack = _stn_stack_kernel
    feat_stack = _feat_stack_kernel
    if num_lt > 1:
        def _accum(body):
            def wrapped(*refs):
                o_ref = refs[-1]

                @pl.when(pl.program_id(1) == 0)
                def _init():
                    o_ref[...] = jnp.full(o_ref.shape, -jnp.inf, o_ref.dtype)

                prev = o_ref[...]
                body(*refs)
                o_ref[...] = jnp.maximum(o_ref[...], prev)
            return wrapped
        stn_stack = _accum(stn_stack)
        feat_stack = _accum(feat_stack)

    g1 = pl.pallas_call(
        stn_stack,
        out_shape=pool_out_shape,
        grid=grid,
        in_specs=[x_spec, _full_spec((64, n)), w2t_spec, w3_spec],
        out_specs=pool_out_spec,
        compiler_params=cparams_pool,
    )(x, w1t_b, w2t_b, w3_b)
    g1 = g1.reshape(B, 1024)

    trans_flat, w1x_eff = pl.pallas_call(
        _stn_head_kernel,
        out_shape=[jax.ShapeDtypeStruct((B, n * n), jnp.float32),
                   jax.ShapeDtypeStruct((B, n, 64), jnp.bfloat16)],
        grid=(1,),
        in_specs=[_full_spec((B, 1024)), _full_spec((1024, 512)),
                  _full_spec((512, 256)), _full_spec((256, n * n)),
                  _full_spec((1, n * n)), _full_spec((n, 64))],
        out_specs=[_full_spec((B, n * n)), _full_spec((B, n, 64))],
        compiler_params=cparams_head,
    )(g1, stn_fw1, stn_fw2, stn_fw3, stn_fb3, f_w1x)
    trans = trans_flat.reshape(B, n, n)

    g2 = pl.pallas_call(
        feat_stack,
        out_shape=pool_out_shape,
        grid=grid,
        in_specs=[x_spec, c_spec, w1x_spec,
                  _full_spec((64, ne)), w2t_spec, w3_spec],
        out_specs=pool_out_spec,
        compiler_params=cparams_pool,
    )(x, c, w1x_eff, fw1ct_b, fw2t_b, fw3_b)
    g2 = g2.reshape(B, 1024)

    logp = pl.pallas_call(
        _cls_head_kernel,
        out_shape=jax.ShapeDtypeStruct((B, k), jnp.float32),
        grid=(1,),
        in_specs=[_full_spec((B, 1024)),
                  _full_spec((1024, 512)), _full_spec((1, 512)),
                  _full_spec((512, 256)), _full_spec((1, 256)),
                  _full_spec((256, k)), _full_spec((1, k))],
        out_specs=_full_spec((B, k)),
        compiler_params=cparams_head,
    )(g2, c_w1, c_b1, c_w2, c_b2, c_w3, c_b3)

    return logp, trans
